# baseline (speedup 1.0000x reference)
import functools
import math

import jax
import jax.numpy as jnp
from jax import lax
from jax.experimental import pallas as pl
from jax.experimental.pallas import tpu as pltpu
from jax.experimental.pallas import tpu_sc as plsc

_NBUF = 3


def _sc_add_pe(x, pe_table):
    B, S, D = x.shape
    info = plsc.get_sparse_core_info()
    NC, NS, L = info.num_cores, info.num_subcores, info.num_lanes
    NW = NC * NS
    assert S % NW == 0
    rows_per_w = S // NW
    CR = 8
    assert rows_per_w % CR == 0
    n_chunks = rows_per_w // CR
    VPC = CR * (D // L)
    scale = math.sqrt(D)

    @functools.partial(
        pl.kernel,
        mesh=plsc.VectorSubcoreMesh(core_axis_name="c", subcore_axis_name="s"),
        out_type=jax.ShapeDtypeStruct((B, S, D), jnp.float32),
        scratch_types=[
            pltpu.VMEM((_NBUF, B, CR, D), jnp.float32),
            pltpu.VMEM((_NBUF, CR, D), jnp.float32),
            [pltpu.SemaphoreType.DMA] * _NBUF,
            [pltpu.SemaphoreType.DMA] * _NBUF,
        ],
        compiler_params=pltpu.CompilerParams(use_tc_tiling_on_sc=True),
    )
    def k(x_hbm, pe_hbm, out_hbm, xbuf, pebuf, isems, osems):
        wid = lax.axis_index("s") * NC + lax.axis_index("c")
        base_row = wid * rows_per_w

        def issue_in(c):
            slot = c % _NBUF
            r0 = pl.multiple_of(base_row + c * CR, CR)
            return [
                pltpu.async_copy(
                    pe_hbm.at[pl.ds(r0, CR), :], pebuf.at[slot], isems[slot]
                ),
                pltpu.async_copy(
                    x_hbm.at[:, pl.ds(r0, CR), :], xbuf.at[slot], isems[slot]
                ),
            ]

        def issue_out(c):
            slot = c % _NBUF
            r0 = pl.multiple_of(base_row + c * CR, CR)
            return [
                pltpu.async_copy(
                    xbuf.at[slot], out_hbm.at[:, pl.ds(r0, CR), :], osems[slot]
                )
            ]

        pending_out = [None] * _NBUF
        pending_in = [None] * _NBUF
        for c in range(_NBUF - 1):
            pending_in[c] = issue_in(c)
        for c in range(n_chunks):
            slot = c % _NBUF
            nxt = c + _NBUF - 1
            if nxt < n_chunks:
                nslot = nxt % _NBUF
                if pending_out[nslot] is not None:
                    for cp in pending_out[nslot]:
                        cp.wait()
                    pending_out[nslot] = None
                pending_in[nslot] = issue_in(nxt)
            for cp in pending_in[slot]:
                cp.wait()
            pending_in[slot] = None

            @plsc.parallel_loop(0, VPC, unroll=8)
            def body(i):
                r = i // (D // L)
                o = pl.multiple_of((i % (D // L)) * L, L)
                vpe = pebuf[slot, r, pl.ds(o, L)] * scale
                for b in range(B):
                    plsc.addupdate(xbuf.at[slot, b, r, pl.ds(o, L)], vpe)

            pending_out[slot] = issue_out(c)
        for po in pending_out:
            if po is not None:
                for cp in po:
                    cp.wait()

    return k(x, pe_table)


def kernel(x, pe_table):
    return _sc_add_pe(x, pe_table)
